# baseline (speedup 1.0000x reference)
import functools

import jax
import jax.numpy as jnp
from jax.experimental import pallas as pl
from jax.experimental.pallas import tpu as pltpu
from jax.experimental.pallas import tpu_sc as plsc

_MESH = plsc.ScalarSubcoreMesh(axis_name="c", num_cores=1)


def _body(x_hbm, w_hbm, b_hbm, out_hbm, xv, wv, bv, ov, sem):
    cx = pltpu.make_async_copy(x_hbm, xv, sem)
    cw = pltpu.make_async_copy(w_hbm, wv, sem)
    cb = pltpu.make_async_copy(b_hbm, bv, sem)
    cx.start()
    cw.start()
    cb.start()
    cx.wait()
    cw.wait()
    cb.wait()

    xs = [xv[i] for i in range(8)]
    flat = (
        xs[4] + 2.0 * xs[6],
        xs[5] + 2.0 * xs[7],
        xs[0],
        xs[1],
        3.0 * xs[6],
        3.0 * xs[7],
    )
    for j in range(4):
        acc = bv[j]
        for k in range(6):
            acc = acc + flat[k] * wv[6 * j + k]
        ov[j] = acc
    pltpu.sync_copy(ov, out_hbm)


@jax.jit
def _run(xf, wf, b):
    k = functools.partial(
        pl.kernel,
        out_type=jax.ShapeDtypeStruct((4,), jnp.float32),
        mesh=_MESH,
        scratch_types=[
            pltpu.SMEM((8,), jnp.float32),
            pltpu.SMEM((24,), jnp.float32),
            pltpu.SMEM((4,), jnp.float32),
            pltpu.SMEM((4,), jnp.float32),
            pltpu.SemaphoreType.DMA,
        ],
        compiler_params=pltpu.CompilerParams(needs_layout_passes=False),
    )(_body)
    return k(xf, wf, b)


def kernel(x, W, b):
    return _run(x.reshape(8), W.reshape(24), b)


# speedup vs baseline: 1.0046x; 1.0046x over previous
import functools

import jax
import jax.numpy as jnp
from jax.experimental import pallas as pl
from jax.experimental.pallas import tpu as pltpu
from jax.experimental.pallas import tpu_sc as plsc

_MESH = plsc.ScalarSubcoreMesh(axis_name="c", num_cores=1)


def _body(p_hbm, out_hbm, pv, ov, sem):
    pltpu.sync_copy(p_hbm, pv)

    xs = [pv[i] for i in range(8)]
    flat = (
        xs[4] + 2.0 * xs[6],
        xs[5] + 2.0 * xs[7],
        xs[0],
        xs[1],
        3.0 * xs[6],
        3.0 * xs[7],
    )
    for j in range(4):
        acc = pv[32 + j]
        for k in range(6):
            acc = acc + flat[k] * pv[8 + 6 * j + k]
        ov[j] = acc
    pltpu.sync_copy(ov, out_hbm)


@jax.jit
def _run(packed):
    k = functools.partial(
        pl.kernel,
        out_type=jax.ShapeDtypeStruct((4,), jnp.float32),
        mesh=_MESH,
        scratch_types=[
            pltpu.SMEM((36,), jnp.float32),
            pltpu.SMEM((4,), jnp.float32),
            pltpu.SemaphoreType.DMA,
        ],
        compiler_params=pltpu.CompilerParams(needs_layout_passes=False),
    )(_body)
    return k(packed)


def kernel(x, W, b):
    return _run(jnp.concatenate([x.reshape(8), W.reshape(24), b]))
